# baseline (speedup 1.0000x reference)
import functools

import jax
import jax.numpy as jnp
from jax import lax
from jax.experimental import pallas as pl
from jax.experimental.pallas import tpu as pltpu
from jax.experimental.pallas import tpu_sc as plsc

DIM = 64
PAD = 128
CB = 1024
ROW_BLOCK = 8192
NSPLIT = 1


def _argmin_body_pad(x_ref, e_ref, ind_ref, pad_ref):
    _argmin_body(x_ref, e_ref, ind_ref)
    e = e_ref[...]
    pad_ref[:, :DIM] = e
    pad_ref[:, DIM:] = jnp.zeros((CB, PAD - DIM), jnp.float32)


def _argmin_body(x_ref, e_ref, ind_ref):
    x = x_ref[...]
    e = e_ref[...]
    xx = jnp.sum(x * x, axis=1, keepdims=True)
    ee = jnp.sum(e * e, axis=1)[None, :]
    e2 = e + e
    xe2 = lax.dot_general(x, e2, (((1,), (1,)), ((), ())),
                          preferred_element_type=jnp.float32)
    d = (xx - xe2) + ee
    m = jnp.min(d, axis=1, keepdims=True)
    iota = lax.broadcasted_iota(jnp.int32, (1, CB), 1).astype(jnp.float32)
    ind_f = jnp.min(jnp.where(d <= m, iota, jnp.float32(2**30)), axis=1)
    ind_ref[...] = ind_f.astype(jnp.int32).reshape(ind_ref.shape)


def _argmin_indices(xf, embed, with_pad):
    n = xf.shape[0]
    nblk = n // ROW_BLOCK
    rows_per_blk = ROW_BLOCK // PAD
    ind_spec = pl.BlockSpec((rows_per_blk, PAD), lambda i: (i, 0))
    ind_shape = jax.ShapeDtypeStruct((n // PAD, PAD), jnp.int32)
    if with_pad:
        out_specs = [ind_spec, pl.BlockSpec((CB, PAD), lambda i: (0, 0))]
        out_shape = [ind_shape, jax.ShapeDtypeStruct((CB, PAD), jnp.float32)]
        body = _argmin_body_pad
    else:
        out_specs, out_shape, body = ind_spec, ind_shape, _argmin_body
    return pl.pallas_call(
        body,
        grid=(nblk,),
        in_specs=[
            pl.BlockSpec((ROW_BLOCK, DIM), lambda i: (i, 0)),
            pl.BlockSpec((CB, DIM), lambda i: (0, 0)),
        ],
        out_specs=out_specs,
        out_shape=out_shape,
    )(xf, embed)


@functools.lru_cache(maxsize=None)
def _sc_gather_fn(batch):
    info = plsc.get_sparse_core_info()
    nc = info.num_cores
    nw = nc * info.num_subcores
    nrow = batch // PAD
    rows_per_w = nrow // nw
    mesh = plsc.VectorSubcoreMesh(core_axis_name="c", subcore_axis_name="s")

    @functools.partial(
        pl.kernel,
        mesh=mesh,
        out_type=jax.ShapeDtypeStruct((nrow, PAD, PAD), jnp.float32),
        scratch_types=[
            pltpu.VMEM((rows_per_w, PAD), jnp.int32),
            pltpu.VMEM((rows_per_w, PAD, PAD), jnp.float32),
            pltpu.SemaphoreType.DMA,
        ],
    )
    def gather(table_hbm, idx_hbm, out_hbm, idx_v, rows_v, sem):
        wid = lax.axis_index("s") * nc + lax.axis_index("c")
        base = wid * rows_per_w
        pltpu.sync_copy(idx_hbm.at[pl.ds(base, rows_per_w)], idx_v)
        copies = [
            pltpu.async_copy(table_hbm.at[idx_v.at[j]], rows_v.at[j], sem)
            for j in range(rows_per_w)
        ]
        for c in copies:
            c.wait()
        pltpu.sync_copy(rows_v, out_hbm.at[pl.ds(base, rows_per_w)])

    return gather


def kernel(x, embed):
    shape = x.shape
    n = x.shape[0] * x.shape[1]
    nh = n // NSPLIT
    xf = x.reshape(-1, shape[-1]).astype(jnp.float32)
    e32 = embed.astype(jnp.float32)

    inds, rows = [], []
    embed_pad = None
    for h in range(NSPLIT):
        xh = xf[h * nh:(h + 1) * nh]
        if h == 0:
            ind_h, embed_pad = _argmin_indices(xh, e32, True)
        else:
            ind_h = _argmin_indices(xh, e32, False)
        inds.append(ind_h)
        rows.append(_sc_gather_fn(nh)(embed_pad, ind_h))

    quantize = jnp.concatenate(
        [r.reshape(nh, PAD)[:, :DIM] for r in rows], axis=0)
    ind = jnp.concatenate(inds, axis=0)
    return (quantize.reshape(shape).astype(x.dtype),
            ind.reshape(shape[:-1]))


# speedup vs baseline: 1.0215x; 1.0215x over previous
import functools

import jax
import jax.numpy as jnp
from jax import lax
from jax.experimental import pallas as pl
from jax.experimental.pallas import tpu as pltpu
from jax.experimental.pallas import tpu_sc as plsc

DIM = 64
PAD = 128
CB = 1024
LANE_BLOCK = 1024
BATCH_BLOCK = 4


def _argmin_body(xt_ref, et_ref, xx_ref, ind_ref, pad_ref):
    et = et_ref[...]
    ee = jnp.sum(et * et, axis=0)[None, :]
    et2 = et + et
    rpb = LANE_BLOCK // PAD
    for bb in range(BATCH_BLOCK):
        xb = xt_ref[bb]
        xx = xx_ref[pl.ds(bb * LANE_BLOCK, LANE_BLOCK), :]
        xe2 = lax.dot_general(xb, et2, (((0,), (0,)), ((), ())),
                              preferred_element_type=jnp.float32)
        d = (xx - xe2) + ee
        m = jnp.min(d, axis=1, keepdims=True)
        iota = lax.broadcasted_iota(jnp.int32, (1, CB), 1).astype(jnp.float32)
        ind_f = jnp.min(jnp.where(d <= m, iota, jnp.float32(2**30)), axis=1)
        ind_ref[pl.ds(bb * rpb, rpb), :] = (
            ind_f.astype(jnp.int32).reshape(rpb, PAD))
    pad_ref[:, :DIM] = et.T
    pad_ref[:, DIM:] = jnp.zeros((CB, PAD - DIM), jnp.float32)


def _argmin_indices(xt, et, xx):
    nb = xt.shape[0]
    n = nb * xt.shape[2]
    grid = nb // BATCH_BLOCK
    rows_per_step = BATCH_BLOCK * LANE_BLOCK // PAD
    ind2d, embed_pad = pl.pallas_call(
        _argmin_body,
        grid=(grid,),
        in_specs=[
            pl.BlockSpec((BATCH_BLOCK, DIM, LANE_BLOCK), lambda i: (i, 0, 0)),
            pl.BlockSpec((DIM, CB), lambda i: (0, 0)),
            pl.BlockSpec((BATCH_BLOCK * LANE_BLOCK, 1), lambda i: (i, 0)),
        ],
        out_specs=[
            pl.BlockSpec((rows_per_step, PAD), lambda i: (i, 0)),
            pl.BlockSpec((CB, PAD), lambda i: (0, 0)),
        ],
        out_shape=[
            jax.ShapeDtypeStruct((n // PAD, PAD), jnp.int32),
            jax.ShapeDtypeStruct((CB, PAD), jnp.float32),
        ],
    )(xt, et, xx)
    return ind2d, embed_pad


@functools.lru_cache(maxsize=None)
def _sc_gather_fn(batch):
    info = plsc.get_sparse_core_info()
    nc = info.num_cores
    nw = nc * info.num_subcores
    nrow = batch // PAD
    rows_per_w = nrow // nw
    mesh = plsc.VectorSubcoreMesh(core_axis_name="c", subcore_axis_name="s")

    @functools.partial(
        pl.kernel,
        mesh=mesh,
        out_type=jax.ShapeDtypeStruct((nrow, PAD, PAD), jnp.float32),
        scratch_types=[
            pltpu.VMEM((rows_per_w, PAD), jnp.int32),
            pltpu.VMEM((rows_per_w, PAD, PAD), jnp.float32),
            pltpu.SemaphoreType.DMA,
        ],
    )
    def gather(table_hbm, idx_hbm, out_hbm, idx_v, rows_v, sem):
        wid = lax.axis_index("s") * nc + lax.axis_index("c")
        base = wid * rows_per_w
        pltpu.sync_copy(idx_hbm.at[pl.ds(base, rows_per_w)], idx_v)
        copies = [
            pltpu.async_copy(table_hbm.at[idx_v.at[j]], rows_v.at[j], sem)
            for j in range(rows_per_w)
        ]
        for c in copies:
            c.wait()
        pltpu.sync_copy(rows_v, out_hbm.at[pl.ds(base, rows_per_w)])

    return gather


def kernel(x, embed):
    shape = x.shape
    n = x.shape[0] * x.shape[1]
    x = x.astype(jnp.float32)
    xt = jnp.transpose(x, (0, 2, 1))
    et = jnp.transpose(embed.astype(jnp.float32))
    xx = jnp.sum(x * x, axis=-1).reshape(n, 1)
    ind2d, embed_pad = _argmin_indices(xt, et, xx)
    rows = _sc_gather_fn(n)(embed_pad, ind2d)
    quantize = rows.reshape(n, PAD)[:, :DIM]
    return (quantize.reshape(shape).astype(x.dtype),
            ind2d.reshape(shape[:-1]))


# speedup vs baseline: 1.1827x; 1.1577x over previous
import functools

import jax
import jax.numpy as jnp
from jax import lax
from jax.experimental import pallas as pl
from jax.experimental.pallas import tpu as pltpu
from jax.experimental.pallas import tpu_sc as plsc

DIM = 64
PAD = 128
CB = 1024
LANE_BLOCK = 1024
BATCH_BLOCK = 4


def _argmin_body(xt_ref, et_ref, ind_ref, pad_ref):
    et = et_ref[...]
    etT = et.T
    ee = jnp.sum(etT * etT, axis=1, keepdims=True)
    et2 = et + et
    rpb = LANE_BLOCK // PAD
    for bb in range(BATCH_BLOCK):
        xb = xt_ref[bb]
        xx = jnp.sum(xb * xb, axis=0)[None, :]
        xe2 = lax.dot_general(et2, xb, (((0,), (0,)), ((), ())),
                              preferred_element_type=jnp.float32)
        d = (xx - xe2) + ee
        m = jnp.min(d, axis=0, keepdims=True)
        iota = lax.broadcasted_iota(jnp.int32, (CB, 1), 0).astype(jnp.float32)
        ind_f = jnp.min(jnp.where(d <= m, iota, jnp.float32(2**30)), axis=0)
        ind_ref[pl.ds(bb * rpb, rpb), :] = (
            ind_f.astype(jnp.int32).reshape(rpb, PAD))
    pad_ref[:, :DIM] = etT
    pad_ref[:, DIM:] = jnp.zeros((CB, PAD - DIM), jnp.float32)


def _argmin_indices(xt, et):
    nb = xt.shape[0]
    n = nb * xt.shape[2]
    grid = nb // BATCH_BLOCK
    rows_per_step = BATCH_BLOCK * LANE_BLOCK // PAD
    ind2d, embed_pad = pl.pallas_call(
        _argmin_body,
        grid=(grid,),
        in_specs=[
            pl.BlockSpec((BATCH_BLOCK, DIM, LANE_BLOCK), lambda i: (i, 0, 0)),
            pl.BlockSpec((DIM, CB), lambda i: (0, 0)),
        ],
        out_specs=[
            pl.BlockSpec((rows_per_step, PAD), lambda i: (i, 0)),
            pl.BlockSpec((CB, PAD), lambda i: (0, 0)),
        ],
        out_shape=[
            jax.ShapeDtypeStruct((n // PAD, PAD), jnp.int32),
            jax.ShapeDtypeStruct((CB, PAD), jnp.float32),
        ],
    )(xt, et)
    return ind2d, embed_pad


@functools.lru_cache(maxsize=None)
def _sc_gather_fn(batch):
    info = plsc.get_sparse_core_info()
    nc = info.num_cores
    nw = nc * info.num_subcores
    nrow = batch // PAD
    rows_per_w = nrow // nw
    mesh = plsc.VectorSubcoreMesh(core_axis_name="c", subcore_axis_name="s")

    @functools.partial(
        pl.kernel,
        mesh=mesh,
        out_type=jax.ShapeDtypeStruct((nrow, PAD, PAD), jnp.float32),
        scratch_types=[
            pltpu.VMEM((rows_per_w, PAD), jnp.int32),
            pltpu.VMEM((rows_per_w, PAD, PAD), jnp.float32),
            pltpu.SemaphoreType.DMA,
        ],
    )
    def gather(table_hbm, idx_hbm, out_hbm, idx_v, rows_v, sem):
        wid = lax.axis_index("s") * nc + lax.axis_index("c")
        base = wid * rows_per_w
        pltpu.sync_copy(idx_hbm.at[pl.ds(base, rows_per_w)], idx_v)
        copies = [
            pltpu.async_copy(table_hbm.at[idx_v.at[j]], rows_v.at[j], sem)
            for j in range(rows_per_w)
        ]
        for c in copies:
            c.wait()
        pltpu.sync_copy(rows_v, out_hbm.at[pl.ds(base, rows_per_w)])

    return gather


def kernel(x, embed):
    shape = x.shape
    n = x.shape[0] * x.shape[1]
    x = x.astype(jnp.float32)
    xt = jnp.transpose(x, (0, 2, 1))
    et = jnp.transpose(embed.astype(jnp.float32))
    ind2d, embed_pad = _argmin_indices(xt, et)
    rows = _sc_gather_fn(n)(embed_pad, ind2d)
    quantize = rows.reshape(n, PAD)[:, :DIM]
    return (quantize.reshape(shape).astype(x.dtype),
            ind2d.reshape(shape[:-1]))


# speedup vs baseline: 1.1833x; 1.0005x over previous
import functools

import jax
import jax.numpy as jnp
from jax import lax
from jax.experimental import pallas as pl
from jax.experimental.pallas import tpu as pltpu
from jax.experimental.pallas import tpu_sc as plsc

DIM = 64
PAD = 128
CB = 1024
LANE_BLOCK = 1024
BATCH_BLOCK = 8


def _argmin_body(xt_ref, et_ref, ind_ref, pad_ref):
    et = et_ref[...]
    etT = et.T
    ee = jnp.sum(etT * etT, axis=1, keepdims=True)
    et2 = et + et
    rpb = LANE_BLOCK // PAD
    for bb in range(BATCH_BLOCK):
        xb = xt_ref[bb]
        xx = jnp.sum(xb * xb, axis=0)[None, :]
        xe2 = lax.dot_general(et2, xb, (((0,), (0,)), ((), ())),
                              preferred_element_type=jnp.float32)
        d = (xx - xe2) + ee
        m = jnp.min(d, axis=0, keepdims=True)
        iota = lax.broadcasted_iota(jnp.int32, (CB, 1), 0).astype(jnp.float32)
        ind_f = jnp.min(jnp.where(d <= m, iota, jnp.float32(2**30)), axis=0)
        ind_ref[pl.ds(bb * rpb, rpb), :] = (
            ind_f.astype(jnp.int32).reshape(rpb, PAD))
    pad_ref[:, :DIM] = etT
    pad_ref[:, DIM:] = jnp.zeros((CB, PAD - DIM), jnp.float32)


def _argmin_indices(xt, et):
    nb = xt.shape[0]
    n = nb * xt.shape[2]
    grid = nb // BATCH_BLOCK
    rows_per_step = BATCH_BLOCK * LANE_BLOCK // PAD
    ind2d, embed_pad = pl.pallas_call(
        _argmin_body,
        grid=(grid,),
        in_specs=[
            pl.BlockSpec((BATCH_BLOCK, DIM, LANE_BLOCK), lambda i: (i, 0, 0)),
            pl.BlockSpec((DIM, CB), lambda i: (0, 0)),
        ],
        out_specs=[
            pl.BlockSpec((rows_per_step, PAD), lambda i: (i, 0)),
            pl.BlockSpec((CB, PAD), lambda i: (0, 0)),
        ],
        out_shape=[
            jax.ShapeDtypeStruct((n // PAD, PAD), jnp.int32),
            jax.ShapeDtypeStruct((CB, PAD), jnp.float32),
        ],
    )(xt, et)
    return ind2d, embed_pad


@functools.lru_cache(maxsize=None)
def _sc_gather_fn(batch):
    info = plsc.get_sparse_core_info()
    nc = info.num_cores
    nw = nc * info.num_subcores
    nrow = batch // PAD
    rows_per_w = nrow // nw
    mesh = plsc.VectorSubcoreMesh(core_axis_name="c", subcore_axis_name="s")

    @functools.partial(
        pl.kernel,
        mesh=mesh,
        out_type=jax.ShapeDtypeStruct((nrow, PAD, PAD), jnp.float32),
        scratch_types=[
            pltpu.VMEM((rows_per_w, PAD), jnp.int32),
            pltpu.VMEM((rows_per_w, PAD, PAD), jnp.float32),
            pltpu.SemaphoreType.DMA,
        ],
    )
    def gather(table_hbm, idx_hbm, out_hbm, idx_v, rows_v, sem):
        wid = lax.axis_index("s") * nc + lax.axis_index("c")
        base = wid * rows_per_w
        pltpu.sync_copy(idx_hbm.at[pl.ds(base, rows_per_w)], idx_v)
        copies = [
            pltpu.async_copy(table_hbm.at[idx_v.at[j]], rows_v.at[j], sem)
            for j in range(rows_per_w)
        ]
        for c in copies:
            c.wait()
        pltpu.sync_copy(rows_v, out_hbm.at[pl.ds(base, rows_per_w)])

    return gather


def kernel(x, embed):
    shape = x.shape
    n = x.shape[0] * x.shape[1]
    x = x.astype(jnp.float32)
    xt = jnp.transpose(x, (0, 2, 1))
    et = jnp.transpose(embed.astype(jnp.float32))
    ind2d, embed_pad = _argmin_indices(xt, et)
    rows = _sc_gather_fn(n)(embed_pad, ind2d)
    quantize = rows.reshape(n, PAD)[:, :DIM]
    return (quantize.reshape(shape).astype(x.dtype),
            ind2d.reshape(shape[:-1]))


# speedup vs baseline: 1.1862x; 1.0025x over previous
import functools

import jax
import jax.numpy as jnp
from jax import lax
from jax.experimental import pallas as pl
from jax.experimental.pallas import tpu as pltpu
from jax.experimental.pallas import tpu_sc as plsc

DIM = 64
PAD = 128
CB = 1024
LANE_BLOCK = 1024
BATCH_BLOCK = 8


def _argmin_body(xt_ref, et_ref, ind_ref, indb_ref, pad_ref):
    et = et_ref[...]
    etT = et.T
    ee = jnp.sum(etT * etT, axis=1, keepdims=True)
    et2 = et + et
    rpb = LANE_BLOCK // PAD
    for bb in range(BATCH_BLOCK):
        xb = xt_ref[bb]
        xx = jnp.sum(xb * xb, axis=0)[None, :]
        xe2 = lax.dot_general(et2, xb, (((0,), (0,)), ((), ())),
                              preferred_element_type=jnp.float32)
        d = (xx - xe2) + ee
        m = jnp.min(d, axis=0, keepdims=True)
        iota = lax.broadcasted_iota(jnp.int32, (CB, 1), 0).astype(jnp.float32)
        ind_f = jnp.min(jnp.where(d <= m, iota, jnp.float32(2**30)), axis=0)
        ind_i = ind_f.astype(jnp.int32)
        ind_ref[pl.ds(bb * rpb, rpb), :] = ind_i.reshape(rpb, PAD)
        indb_ref[pl.ds(bb, 1), :] = ind_i.reshape(1, LANE_BLOCK)
    pad_ref[:, :DIM] = etT
    pad_ref[:, DIM:] = jnp.zeros((CB, PAD - DIM), jnp.float32)


def _argmin_indices(xt, et):
    nb = xt.shape[0]
    n = nb * xt.shape[2]
    grid = nb // BATCH_BLOCK
    rows_per_step = BATCH_BLOCK * LANE_BLOCK // PAD
    ind2d, indb, embed_pad = pl.pallas_call(
        _argmin_body,
        grid=(grid,),
        in_specs=[
            pl.BlockSpec((BATCH_BLOCK, DIM, LANE_BLOCK), lambda i: (i, 0, 0)),
            pl.BlockSpec((DIM, CB), lambda i: (0, 0)),
        ],
        out_specs=[
            pl.BlockSpec((rows_per_step, PAD), lambda i: (i, 0)),
            pl.BlockSpec((BATCH_BLOCK, LANE_BLOCK), lambda i: (i, 0)),
            pl.BlockSpec((CB, PAD), lambda i: (0, 0)),
        ],
        out_shape=[
            jax.ShapeDtypeStruct((n // PAD, PAD), jnp.int32),
            jax.ShapeDtypeStruct((nb, LANE_BLOCK), jnp.int32),
            jax.ShapeDtypeStruct((CB, PAD), jnp.float32),
        ],
    )(xt, et)
    return ind2d, indb, embed_pad


@functools.lru_cache(maxsize=None)
def _sc_gather_fn(batch):
    info = plsc.get_sparse_core_info()
    nc = info.num_cores
    nw = nc * info.num_subcores
    nrow = batch // PAD
    rows_per_w = nrow // nw
    mesh = plsc.VectorSubcoreMesh(core_axis_name="c", subcore_axis_name="s")

    @functools.partial(
        pl.kernel,
        mesh=mesh,
        out_type=jax.ShapeDtypeStruct((nrow, PAD, PAD), jnp.float32),
        scratch_types=[
            pltpu.VMEM((rows_per_w, PAD), jnp.int32),
            pltpu.VMEM((rows_per_w, PAD, PAD), jnp.float32),
            pltpu.SemaphoreType.DMA,
        ],
    )
    def gather(table_hbm, idx_hbm, out_hbm, idx_v, rows_v, sem):
        wid = lax.axis_index("s") * nc + lax.axis_index("c")
        base = wid * rows_per_w
        pltpu.sync_copy(idx_hbm.at[pl.ds(base, rows_per_w)], idx_v)
        copies = [
            pltpu.async_copy(table_hbm.at[idx_v.at[j]], rows_v.at[j], sem)
            for j in range(rows_per_w)
        ]
        for c in copies:
            c.wait()
        pltpu.sync_copy(rows_v, out_hbm.at[pl.ds(base, rows_per_w)])

    return gather


def kernel(x, embed):
    shape = x.shape
    n = x.shape[0] * x.shape[1]
    x = x.astype(jnp.float32)
    xt = jnp.transpose(x, (0, 2, 1))
    et = jnp.transpose(embed.astype(jnp.float32))
    ind2d, indb, embed_pad = _argmin_indices(xt, et)
    rows = _sc_gather_fn(n)(embed_pad, ind2d)
    quantize = rows.reshape(n, PAD)[:, :DIM]
    return (quantize.reshape(shape).astype(x.dtype),
            indb.reshape(shape[:-1]))
